# baseline (speedup 1.0000x reference)
import functools

import jax
import jax.numpy as jnp
from jax import lax
from jax.experimental import pallas as pl
from jax.experimental.pallas import tpu as pltpu
from jax.experimental.pallas import tpu_sc as plsc

_B = 16384
_DIM = 32
_NC = 2
_NS = 16
_NW = _NC * _NS
_BPW = _B // _NW
_CH = 128
_NCH = _BPW // _CH


def _body(user_idx, item_idx, time_idx, user_tab, item_tab, time_tab, out,
          idx_v, rows_u, rows_i, rows_t, sem):
    wid = lax.axis_index("s") * _NC + lax.axis_index("c")
    base = wid * _BPW
    cbase = wid * _NCH

    pltpu.sync_copy(user_idx.at[pl.ds(cbase, _NCH)], idx_v.at[0])
    pltpu.sync_copy(item_idx.at[pl.ds(cbase, _NCH)], idx_v.at[1])
    pltpu.sync_copy(time_idx.at[pl.ds(cbase, _NCH)], idx_v.at[2])

    copies = []
    for j in range(_NCH):
        copies.append(pltpu.async_copy(user_tab.at[idx_v.at[0, j]], rows_u.at[j], sem))
        copies.append(pltpu.async_copy(item_tab.at[idx_v.at[1, j]], rows_i.at[j], sem))
        copies.append(pltpu.async_copy(time_tab.at[idx_v.at[2, j]], rows_t.at[j], sem))
    for c in copies:
        c.wait()

    wcopies = []
    for j in range(_NCH):
        dst_rows = pl.ds(base + j * _CH, _CH)
        wcopies.append(pltpu.async_copy(rows_u.at[j], out.at[dst_rows, pl.ds(0, _DIM)], sem))
        wcopies.append(pltpu.async_copy(rows_i.at[j], out.at[dst_rows, pl.ds(_DIM, _DIM)], sem))
        wcopies.append(pltpu.async_copy(rows_t.at[j], out.at[dst_rows, pl.ds(2 * _DIM, _DIM)], sem))
    for c in wcopies:
        c.wait()


_emb_call = functools.partial(
    pl.kernel,
    out_type=jax.ShapeDtypeStruct((_B, 3 * _DIM), jnp.float32),
    mesh=plsc.VectorSubcoreMesh(core_axis_name="c", subcore_axis_name="s"),
    scratch_types=[
        pltpu.VMEM((3, _NCH, _CH), jnp.int32),
        pltpu.VMEM((_NCH, _CH, _DIM), jnp.float32),
        pltpu.VMEM((_NCH, _CH, _DIM), jnp.float32),
        pltpu.VMEM((_NCH, _CH, _DIM), jnp.float32),
        pltpu.SemaphoreType.DMA,
    ],
    compiler_params=pltpu.CompilerParams(use_tc_tiling_on_sc=False),
)(_body)


@jax.jit
def kernel(user_idx, item_idx, time_idx, user_table, item_table, time_table):
    u2 = user_idx.reshape(_B // _CH, _CH)
    i2 = item_idx.reshape(_B // _CH, _CH)
    t2 = time_idx.reshape(_B // _CH, _CH)
    return _emb_call(u2, i2, t2, user_table, item_table, time_table)
